# Initial kernel scaffold; baseline (speedup 1.0000x reference)
import jax
import jax.numpy as jnp
from jax.experimental import pallas as pl


def kernel(input, table):
    raise NotImplementedError("write your pallas kernel here")


# speedup vs baseline: 2.9657x; 2.9657x over previous
import functools

import jax
import jax.numpy as jnp
from jax import lax
from jax.experimental import pallas as pl
from jax.experimental.pallas import tpu as pltpu
from jax.experimental.pallas import tpu_sc as plsc

D = 128
NC = 2
NS = 16
NW = NC * NS
B_TOTAL = 4096 * 50
B_PER_W = B_TOTAL // NW
CHUNK = 128
NCHUNK = B_PER_W // CHUNK

_MESH = plsc.VectorSubcoreMesh(core_axis_name="c", subcore_axis_name="s")


@functools.partial(
    pl.kernel,
    mesh=_MESH,
    out_type=jax.ShapeDtypeStruct((B_TOTAL, D), jnp.float32),
    scratch_types=[
        pltpu.VMEM((NCHUNK, CHUNK), jnp.int32),
        pltpu.VMEM((CHUNK, D), jnp.float32),
        pltpu.SemaphoreType.DMA,
    ],
)
def _embed_gather(idx_hbm, table_hbm, out_hbm, idx_v, rows_v, sem):
    wid = lax.axis_index("s") * NC + lax.axis_index("c")
    base = wid * B_PER_W
    pltpu.sync_copy(idx_hbm.at[wid], idx_v)

    def body(g, carry):
        pltpu.async_copy(table_hbm.at[idx_v.at[g]], rows_v, sem).wait()
        pltpu.sync_copy(rows_v, out_hbm.at[pl.ds(base + g * CHUNK, CHUNK)])
        return carry

    lax.fori_loop(0, NCHUNK, body, 0)


def kernel(input, table):
    batch, hist = input.shape
    idx = input.reshape(NW, NCHUNK, CHUNK).astype(jnp.int32)
    out = _embed_gather(idx, table)
    return out.reshape(batch, hist, D)


# speedup vs baseline: 3.3294x; 1.1227x over previous
import functools

import jax
import jax.numpy as jnp
from jax import lax
from jax.experimental import pallas as pl
from jax.experimental.pallas import tpu as pltpu
from jax.experimental.pallas import tpu_sc as plsc

D = 128
NC = 2
NS = 16
NW = NC * NS
B_TOTAL = 4096 * 50
B_PER_W = B_TOTAL // NW
CHUNK = 128
NCHUNK = B_PER_W // CHUNK

_MESH = plsc.VectorSubcoreMesh(core_axis_name="c", subcore_axis_name="s")


@functools.partial(
    pl.kernel,
    mesh=_MESH,
    out_type=jax.ShapeDtypeStruct((B_TOTAL, D), jnp.float32),
    scratch_types=[
        pltpu.VMEM((NCHUNK, CHUNK), jnp.int32),
        pltpu.VMEM((CHUNK, D), jnp.float32),
        pltpu.VMEM((CHUNK, D), jnp.float32),
        pltpu.SemaphoreType.DMA,
        pltpu.SemaphoreType.DMA,
        pltpu.SemaphoreType.DMA,
        pltpu.SemaphoreType.DMA,
    ],
)
def _embed_gather(idx_hbm, table_hbm, out_hbm, idx_v, rows0, rows1,
                  gsem0, gsem1, ssem0, ssem1):
    wid = lax.axis_index("s") * NC + lax.axis_index("c")
    base = wid * B_PER_W
    pltpu.sync_copy(idx_hbm.at[wid], idx_v)

    pltpu.async_copy(table_hbm.at[idx_v.at[0]], rows0, gsem0)

    def body(g2, carry):
        c0 = 2 * g2

        @pl.when(g2 > 0)
        def _():
            pltpu.make_async_copy(
                rows1, out_hbm.at[pl.ds(base, CHUNK)], ssem1).wait()

        g1 = pltpu.async_copy(table_hbm.at[idx_v.at[c0 + 1]], rows1, gsem1)
        pltpu.make_async_copy(
            table_hbm.at[idx_v.at[c0]], rows0, gsem0).wait()
        s0 = pltpu.async_copy(
            rows0, out_hbm.at[pl.ds(base + c0 * CHUNK, CHUNK)], ssem0)

        s0.wait()

        @pl.when(g2 < NCHUNK // 2 - 1)
        def _():
            pltpu.async_copy(table_hbm.at[idx_v.at[c0 + 2]], rows0, gsem0)

        g1.wait()
        pltpu.async_copy(
            rows1, out_hbm.at[pl.ds(base + (c0 + 1) * CHUNK, CHUNK)], ssem1)
        return carry

    lax.fori_loop(0, NCHUNK // 2, body, 0)

    pltpu.make_async_copy(rows1, out_hbm.at[pl.ds(base, CHUNK)], ssem1).wait()


def kernel(input, table):
    batch, hist = input.shape
    idx = input.reshape(NW, NCHUNK, CHUNK).astype(jnp.int32)
    out = _embed_gather(idx, table)
    return out.reshape(batch, hist, D)
